# baseline (speedup 1.0000x reference)
import functools

import jax
import jax.numpy as jnp
from jax import lax
from jax.experimental import pallas as pl
from jax.experimental.pallas import tpu as pltpu
from jax.experimental.pallas import tpu_sc as plsc

N_NODES = 100000
DIM = 64

NC = 2
NS = 16
NW = NC * NS
LANES = 16

CHUNK = 128
N_FULL = N_NODES // CHUNK
N_ROUND = N_FULL // NW
LEFT = N_FULL - N_ROUND * NW
CLIP = N_NODES - N_FULL * CHUNK

_mesh = plsc.VectorSubcoreMesh(core_axis_name="c", subcore_axis_name="s")


@functools.partial(
    pl.kernel,
    mesh=_mesh,
    compiler_params=pltpu.CompilerParams(
        use_tc_tiling_on_sc=True, needs_layout_passes=False),
    out_type=jax.ShapeDtypeStruct((DIM, N_NODES), jnp.float32),
    scratch_types=[
        pltpu.VMEM((CHUNK,), jnp.int32),
        pltpu.VMEM((CHUNK,), jnp.int32),
        pltpu.VMEM((CHUNK,), jnp.int32),
        pltpu.VMEM((CHUNK, 2 * DIM), jnp.float32),
        pltpu.VMEM((8, 8, CHUNK), jnp.float32),
        pltpu.SemaphoreType.DMA,
    ],
)
def _gather_kernel(idx_hbm, tab2_hbm, outT_hbm,
                   idxc, pidx, hbit, rows_v, tbufs, gsem):
    wid = lax.axis_index("s") * NC + lax.axis_index("c")

    def do_chunk(k, width):
        co = pl.multiple_of(k * CHUNK, 128)
        pltpu.sync_copy(idx_hbm.at[pl.ds(co, width)], idxc.at[pl.ds(0, width)])
        for t in range(width // LANES):
            v = idxc[pl.ds(t * LANES, LANES)]
            pidx[pl.ds(t * LANES, LANES)] = lax.shift_right_logical(v, 1)
            hbit[pl.ds(t * LANES, LANES)] = lax.bitwise_and(v, 1)
        pltpu.async_copy(
            tab2_hbm.at[pidx.at[pl.ds(0, width)]],
            rows_v.at[pl.ds(0, width)], gsem,
        ).wait()

        def trans_q(q, _):
            g = q // 8
            qq = lax.rem(q, 8)
            for t in range(width // LANES):
                ridx = t * LANES + lax.iota(jnp.int32, LANES)
                cidx = hbit[pl.ds(t * LANES, LANES)] * DIM + q
                tbufs[g, qq, pl.ds(t * LANES, LANES)] = plsc.load_gather(
                    rows_v, [ridx, cidx])
            return 0

        lax.fori_loop(0, DIM, trans_q, 0)

        for g in range(8):
            pltpu.sync_copy(
                tbufs.at[g, :, pl.ds(0, width)],
                outT_hbm.at[pl.ds(8 * g, 8), pl.ds(co, width)],
            )

    def round_body(c, _):
        do_chunk(wid + NW * c, CHUNK)
        return 0

    lax.fori_loop(0, N_ROUND, round_body, 0)

    @pl.when(wid < LEFT)
    def _():
        do_chunk(N_ROUND * NW + wid, CHUNK)

    @pl.when(wid == LEFT)
    def _():
        do_chunk(N_FULL, CLIP)


def kernel(x, spatial_indexs, table):
    idx = spatial_indexs.astype(jnp.int32)
    tab2 = table.reshape(N_NODES // 2, 2 * DIM)
    outT = _gather_kernel(idx, tab2)
    return outT.T[None, None]


# speedup vs baseline: 1.5099x; 1.5099x over previous
import functools

import jax
import jax.numpy as jnp
from jax import lax
from jax.experimental import pallas as pl
from jax.experimental.pallas import tpu as pltpu
from jax.experimental.pallas import tpu_sc as plsc

N_NODES = 100000
DIM = 64

NC = 2
NS = 16
NW = NC * NS
LANES = 16

CHUNK = 128
N_FULL = N_NODES // CHUNK
N_ROUND = 25
CLIP = N_NODES - N_FULL * CHUNK
CLIP_W = N_FULL % NW

_mesh = plsc.VectorSubcoreMesh(core_axis_name="c", subcore_axis_name="s")


@functools.partial(
    pl.kernel,
    mesh=_mesh,
    compiler_params=pltpu.CompilerParams(
        use_tc_tiling_on_sc=True, needs_layout_passes=False),
    out_type=[
        jax.ShapeDtypeStruct((DIM, N_NODES), jnp.float32),
        jax.ShapeDtypeStruct((CLIP * DIM,), jnp.float32),
    ],
    scratch_types=[
        pltpu.VMEM((2, CHUNK), jnp.int32),
        pltpu.VMEM((2, CHUNK), jnp.int32),
        pltpu.VMEM((2, CHUNK), jnp.int32),
        pltpu.VMEM((2, CHUNK, 2 * DIM), jnp.float32),
        pltpu.VMEM((2, 8, 8, CHUNK), jnp.float32),
        pltpu.VMEM((CLIP * DIM,), jnp.float32),
        pltpu.SemaphoreType.DMA,
        pltpu.SemaphoreType.DMA,
    ],
)
def _gather_kernel(idx_hbm, tab2_hbm, outT_hbm, clip_hbm,
                   idxc, pidx, hbit, rows_v, tbufs, clip_v, gsem, wsem):
    wid = lax.axis_index("s") * NC + lax.axis_index("c")

    def stage_pre(k, b):
        co = pl.multiple_of(k * CHUNK, 128)
        pltpu.sync_copy(idx_hbm.at[pl.ds(co, CHUNK)], idxc.at[b])
        for t in range(CHUNK // LANES):
            v = idxc[b, pl.ds(t * LANES, LANES)]
            pidx[b, pl.ds(t * LANES, LANES)] = lax.shift_right_logical(v, 1)
            hbit[b, pl.ds(t * LANES, LANES)] = lax.bitwise_and(v, 1)
        pltpu.async_copy(tab2_hbm.at[pidx.at[b]], rows_v.at[b], gsem)

    def transpose(b, width):
        nt = width // LANES
        rid = [t * LANES + lax.iota(jnp.int32, LANES) for t in range(nt)]
        cb = [hbit[b, pl.ds(t * LANES, LANES)] * DIM for t in range(nt)]
        src = rows_v.at[b]
        for q in range(DIM):
            for t in range(nt):
                tbufs[b, q // 8, q % 8, pl.ds(t * LANES, LANES)] = (
                    plsc.load_gather(src, [rid[t], cb[t] + q]))

    def out_descs(b, co, sem):
        return [
            pltpu.make_async_copy(
                tbufs.at[b, g],
                outT_hbm.at[pl.ds(8 * g, 8), pl.ds(co, CHUNK)], sem)
            for g in range(8)
        ]

    stage_pre(wid, 0)

    def round_body(c, _):
        k = wid + NW * c
        b = lax.rem(c, 2)
        co = pl.multiple_of(k * CHUNK, 128)

        @pl.when(c >= 2)
        def _():
            for d in out_descs(b, pl.multiple_of(co - 2 * NW * CHUNK, 128), wsem):
                d.wait()

        @pl.when(k < N_FULL)
        def _():
            pltpu.make_async_copy(
                tab2_hbm.at[pidx.at[b]], rows_v.at[b], gsem).wait()

            @pl.when(wid + NW * (c + 1) < N_FULL)
            def _():
                stage_pre(wid + NW * (c + 1), 1 - b)

            transpose(b, CHUNK)
            for d in out_descs(b, co, wsem):
                d.start()

        return 0

    lax.fori_loop(0, N_ROUND, round_body, 0)

    @pl.when(wid + NW * 23 < N_FULL)
    def _():
        for d in out_descs(1, pl.multiple_of((wid + NW * 23) * CHUNK, 128), wsem):
            d.wait()

    @pl.when(wid + NW * 24 < N_FULL)
    def _():
        for d in out_descs(0, pl.multiple_of((wid + NW * 24) * CHUNK, 128), wsem):
            d.wait()

    @pl.when(wid == CLIP_W)
    def _():
        co = N_FULL * CHUNK
        pltpu.sync_copy(idx_hbm.at[pl.ds(co, CLIP)], idxc.at[0, pl.ds(0, CLIP)])
        for t in range(CLIP // LANES):
            v = idxc[0, pl.ds(t * LANES, LANES)]
            pidx[0, pl.ds(t * LANES, LANES)] = lax.shift_right_logical(v, 1)
            hbit[0, pl.ds(t * LANES, LANES)] = lax.bitwise_and(v, 1)
        pltpu.async_copy(
            tab2_hbm.at[pidx.at[0, pl.ds(0, CLIP)]],
            rows_v.at[0, pl.ds(0, CLIP)], gsem).wait()
        src = rows_v.at[0]
        for t in range(CLIP // LANES):
            rid = t * LANES + lax.iota(jnp.int32, LANES)
            cb = hbit[0, pl.ds(t * LANES, LANES)] * DIM
            for q in range(DIM):
                plsc.store_scatter(clip_v, [rid * DIM + q],
                                   plsc.load_gather(src, [rid, cb + q]))
        pltpu.sync_copy(clip_v, clip_hbm)


def kernel(x, spatial_indexs, table):
    idx = spatial_indexs.astype(jnp.int32)
    tab2 = table.reshape(N_NODES // 2, 2 * DIM)
    outT, clip_lin = _gather_kernel(idx, tab2)
    out = outT.T[None, None]
    clip = clip_lin.reshape(1, 1, CLIP, DIM)
    return lax.dynamic_update_slice(out, clip, (0, 0, N_FULL * CHUNK, 0))


# speedup vs baseline: 1.5117x; 1.0012x over previous
import functools

import jax
import jax.numpy as jnp
from jax import lax
from jax.experimental import pallas as pl
from jax.experimental.pallas import tpu as pltpu
from jax.experimental.pallas import tpu_sc as plsc

N_NODES = 100000
DIM = 64

NC = 2
NS = 16
NW = NC * NS
LANES = 16

CHUNK = 128
N_FULL = N_NODES // CHUNK
N_ROUND = 25
CLIP = N_NODES - N_FULL * CHUNK
CLIP_W = N_FULL % NW

_mesh = plsc.VectorSubcoreMesh(core_axis_name="c", subcore_axis_name="s")


@functools.partial(
    pl.kernel,
    mesh=_mesh,
    compiler_params=pltpu.CompilerParams(
        use_tc_tiling_on_sc=True, needs_layout_passes=False),
    out_type=[
        jax.ShapeDtypeStruct((DIM, N_NODES), jnp.float32),
        jax.ShapeDtypeStruct((CLIP * DIM,), jnp.float32),
    ],
    scratch_types=[
        pltpu.VMEM((2, CHUNK), jnp.int32),
        pltpu.VMEM((2, CHUNK), jnp.int32),
        pltpu.VMEM((2, CHUNK), jnp.int32),
        pltpu.VMEM((2, CHUNK, 2 * DIM), jnp.float32),
        pltpu.VMEM((8, 8, CHUNK), jnp.float32),
        pltpu.VMEM((CLIP * DIM,), jnp.float32),
        pltpu.SemaphoreType.DMA,
        pltpu.SemaphoreType.DMA,
    ],
)
def _gather_kernel(idx_hbm, tab2_hbm, outT_hbm, clip_hbm,
                   idxc, pidx, hbit, rows_v, tbufs, clip_v, gsem, wsem):
    wid = lax.axis_index("s") * NC + lax.axis_index("c")

    def stage_pre(k, b):
        co = pl.multiple_of(k * CHUNK, 128)
        pltpu.sync_copy(idx_hbm.at[pl.ds(co, CHUNK)], idxc.at[b])
        for t in range(CHUNK // LANES):
            v = idxc[b, pl.ds(t * LANES, LANES)]
            pidx[b, pl.ds(t * LANES, LANES)] = lax.shift_right_logical(v, 1)
            hbit[b, pl.ds(t * LANES, LANES)] = lax.bitwise_and(v, 1)
        pltpu.async_copy(tab2_hbm.at[pidx.at[b]], rows_v.at[b], gsem)

    def transpose(b):
        nt = CHUNK // LANES
        rid = [t * LANES + lax.iota(jnp.int32, LANES) for t in range(nt)]
        cb = [hbit[b, pl.ds(t * LANES, LANES)] * DIM for t in range(nt)]
        src = rows_v.at[b]
        for q in range(DIM):
            for t in range(nt):
                tbufs[q // 8, q % 8, pl.ds(t * LANES, LANES)] = (
                    plsc.load_gather(src, [rid[t], cb[t] + q]))

    def out_descs(co, sem):
        return [
            pltpu.make_async_copy(
                tbufs.at[g],
                outT_hbm.at[pl.ds(8 * g, 8), pl.ds(co, CHUNK)], sem)
            for g in range(8)
        ]

    stage_pre(wid, 0)

    def round_body(c, _):
        k = wid + NW * c
        b = lax.rem(c, 2)
        co = pl.multiple_of(k * CHUNK, 128)

        @pl.when(k < N_FULL)
        def _():
            pltpu.make_async_copy(
                tab2_hbm.at[pidx.at[b]], rows_v.at[b], gsem).wait()

            @pl.when(wid + NW * (c + 1) < N_FULL)
            def _():
                stage_pre(wid + NW * (c + 1), 1 - b)

            @pl.when(c >= 1)
            def _():
                for d in out_descs(pl.multiple_of(co - NW * CHUNK, 128), wsem):
                    d.wait()

            @pl.when(b == 0)
            def _():
                transpose(0)

            @pl.when(b == 1)
            def _():
                transpose(1)

            for d in out_descs(co, wsem):
                d.start()

        return 0

    lax.fori_loop(0, N_ROUND, round_body, 0)

    @pl.when(wid + NW * 24 < N_FULL)
    def _():
        for d in out_descs(pl.multiple_of((wid + NW * 24) * CHUNK, 128), wsem):
            d.wait()

    @pl.when(wid + NW * 24 >= N_FULL)
    def _():
        for d in out_descs(pl.multiple_of((wid + NW * 23) * CHUNK, 128), wsem):
            d.wait()

    @pl.when(wid == CLIP_W)
    def _():
        co = N_FULL * CHUNK
        pltpu.sync_copy(idx_hbm.at[pl.ds(co, CLIP)], idxc.at[0, pl.ds(0, CLIP)])
        for t in range(CLIP // LANES):
            v = idxc[0, pl.ds(t * LANES, LANES)]
            pidx[0, pl.ds(t * LANES, LANES)] = lax.shift_right_logical(v, 1)
            hbit[0, pl.ds(t * LANES, LANES)] = lax.bitwise_and(v, 1)
        pltpu.async_copy(
            tab2_hbm.at[pidx.at[0, pl.ds(0, CLIP)]],
            rows_v.at[0, pl.ds(0, CLIP)], gsem).wait()
        src = rows_v.at[0]
        for t in range(CLIP // LANES):
            rid = t * LANES + lax.iota(jnp.int32, LANES)
            cb = hbit[0, pl.ds(t * LANES, LANES)] * DIM
            for q in range(DIM):
                plsc.store_scatter(clip_v, [rid * DIM + q],
                                   plsc.load_gather(src, [rid, cb + q]))
        pltpu.sync_copy(clip_v, clip_hbm)


def kernel(x, spatial_indexs, table):
    idx = spatial_indexs.astype(jnp.int32)
    tab2 = table.reshape(N_NODES // 2, 2 * DIM)
    outT, clip_lin = _gather_kernel(idx, tab2)
    out = outT.T[None, None]
    clip = clip_lin.reshape(1, 1, CLIP, DIM)
    return lax.dynamic_update_slice(out, clip, (0, 0, N_FULL * CHUNK, 0))


# speedup vs baseline: 2.3940x; 1.5837x over previous
import functools

import jax
import jax.numpy as jnp
from jax import lax
from jax.experimental import pallas as pl
from jax.experimental.pallas import tpu as pltpu
from jax.experimental.pallas import tpu_sc as plsc

N_NODES = 100000
DIM = 64

NC = 2
NS = 16
NW = NC * NS

CHUNK = 125
N_CHUNKS = 25
PER_W = CHUNK * N_CHUNKS
NBUF = 4

_mesh = plsc.VectorSubcoreMesh(core_axis_name="c", subcore_axis_name="s")


@functools.partial(
    pl.kernel,
    mesh=_mesh,
    compiler_params=pltpu.CompilerParams(use_tc_tiling_on_sc=False),
    out_type=jax.ShapeDtypeStruct((N_NODES, DIM), jnp.float32),
    scratch_types=[
        pltpu.VMEM((N_CHUNKS, CHUNK), jnp.int32),
        pltpu.VMEM((NBUF, CHUNK, DIM), jnp.float32),
        pltpu.SemaphoreType.DMA,
        pltpu.SemaphoreType.DMA,
    ],
)
def _gather_kernel(idx_hbm, table_hbm, out_hbm, idx_v, rows_v, gsem, wsem):
    wid = lax.axis_index("s") * NC + lax.axis_index("c")
    base = wid * PER_W
    pltpu.sync_copy(idx_hbm.at[wid], idx_v)

    for b in range(NBUF):
        pltpu.async_copy(table_hbm.at[idx_v.at[b]], rows_v.at[b], gsem)

    def body(c, _):
        buf = lax.rem(c, NBUF)
        pltpu.make_async_copy(
            table_hbm.at[idx_v.at[c]], rows_v.at[buf], gsem
        ).wait()
        wr = pltpu.async_copy(
            rows_v.at[buf], out_hbm.at[pl.ds(base + c * CHUNK, CHUNK)], wsem
        )

        @pl.when(c + NBUF < N_CHUNKS)
        def _():
            wr.wait()
            pltpu.async_copy(
                table_hbm.at[idx_v.at[c + NBUF]], rows_v.at[buf], gsem
            )

        return 0

    lax.fori_loop(0, N_CHUNKS, body, 0)

    for b in range(NBUF):
        c = N_CHUNKS - NBUF + b
        pltpu.make_async_copy(
            rows_v.at[c % NBUF], out_hbm.at[pl.ds(base + c * CHUNK, CHUNK)], wsem
        ).wait()


def kernel(x, spatial_indexs, table):
    idx3 = spatial_indexs.astype(jnp.int32).reshape(NW, N_CHUNKS, CHUNK)
    out = _gather_kernel(idx3, table)
    return out[None, None]
